# baseline (speedup 1.0000x reference)
import functools

import jax
import jax.numpy as jnp
from jax import lax
from jax.experimental import pallas as pl
from jax.experimental.pallas import tpu as pltpu
from jax.experimental.pallas import tpu_sc as plsc

VOCAB_SIZE = 1000
N_ROWS = 4096 * 26
NUM_WORKERS = 32
ROWS_PER_WORKER = N_ROWS // NUM_WORKERS
RPB = 32
NBLK = ROWS_PER_WORKER // RPB
NBUF = 2
BLOCK_WORDS = RPB * VOCAB_SIZE

_mesh = plsc.VectorSubcoreMesh(core_axis_name="c", subcore_axis_name="s")


@functools.partial(
    pl.kernel,
    out_type=jax.ShapeDtypeStruct((N_ROWS * VOCAB_SIZE,), jnp.float32),
    mesh=_mesh,
    scratch_types=[
        pltpu.VMEM((ROWS_PER_WORKER,), jnp.int32),
        pltpu.VMEM((NBUF * BLOCK_WORDS,), jnp.float32),
        pltpu.SemaphoreType.DMA,
        pltpu.SemaphoreType.DMA,
    ],
    compiler_params=pltpu.CompilerParams(needs_layout_passes=False),
)
def _one_hot_sc(idx_hbm, out_hbm, idx_v, buf_v, sem0, sem1):
    wid = lax.axis_index("c") * 16 + lax.axis_index("s")
    row0 = wid * ROWS_PER_WORKER

    pltpu.sync_copy(idx_hbm.at[pl.ds(row0, ROWS_PER_WORKER)], idx_v)

    zeros16 = jnp.zeros((16,), jnp.float32)

    def _zero(i, _):
        buf_v[pl.ds(i * 16, 16)] = zeros16
        return 0

    lax.fori_loop(0, NBUF * BLOCK_WORDS // 16, _zero, 0, unroll=8)

    lane = jax.lax.iota(jnp.int32, 16)

    def scatter_block(b, slot, value):
        vals16 = jnp.full((16,), value, jnp.float32)
        for c in range(RPB // 16):
            v = idx_v[pl.ds(b * RPB + c * 16, 16)]
            pos = (slot * BLOCK_WORDS + (c * 16) * VOCAB_SIZE) + lane * VOCAB_SIZE + v
            plsc.store_scatter(buf_v, [pos], vals16)

    def dma_out(b, slot, sem):
        return pltpu.async_copy(
            buf_v.at[pl.ds(slot * BLOCK_WORDS, BLOCK_WORDS)],
            out_hbm.at[pl.ds((row0 // RPB + b) * BLOCK_WORDS, BLOCK_WORDS)],
            sem,
        )

    def dma_wait(b, slot, sem):
        pltpu.make_async_copy(
            buf_v.at[pl.ds(slot * BLOCK_WORDS, BLOCK_WORDS)],
            out_hbm.at[pl.ds((row0 // RPB + b) * BLOCK_WORDS, BLOCK_WORDS)],
            sem,
        ).wait()

    sems = (sem0, sem1)

    for ib in range(NBUF):
        scatter_block(ib, ib, 1.0)
        dma_out(ib, ib, sems[ib])

    def body(g, _):
        for ib in range(NBUF):
            b = g * NBUF + ib
            dma_wait(b - NBUF, ib, sems[ib])
            scatter_block(b - NBUF, ib, 0.0)
            scatter_block(b, ib, 1.0)
            dma_out(b, ib, sems[ib])
        return 0

    lax.fori_loop(1, NBLK // NBUF, body, 0)

    for ib in range(NBUF):
        dma_wait(NBLK - NBUF + ib, ib, sems[ib])


def kernel(x):
    idx = x.astype(jnp.int32).reshape(-1)
    out = _one_hot_sc(idx)
    return out.reshape(4096, 26, VOCAB_SIZE)
